# baseline (speedup 1.0000x reference)
import math

import numpy as np
import jax
import jax.numpy as jnp
from jax import lax
from jax.experimental import pallas as pl
from jax.experimental.pallas import tpu as pltpu
from jax.experimental.pallas import tpu_sc as plsc

VOCAB = 32000
D = 2048
BATCH = 4
SEQ = 2048
N = BATCH * SEQ
SCALE = math.sqrt(float(D))

NC = 2
NS = 16
NW = NC * NS
PPW = SEQ // NW
CH = 16
PPC = CH // BATCH
NCH = PPW // PPC


def _sinusoidal_pe_np(seq_len, d_model):
    pos = np.arange(seq_len, dtype=np.float64)[:, None]
    i = np.arange(0, d_model, 2, dtype=np.float64)[None, :]
    angle = pos / np.power(10000.0, i / d_model)
    pe = np.zeros((seq_len, d_model), dtype=np.float32)
    pe[:, 0::2] = np.sin(angle)
    pe[:, 1::2] = np.cos(angle)
    return pe


_PE = _sinusoidal_pe_np(SEQ, D)


def _fused_scale_add(rows, pe):
    def row_body(r, carry):
        pr = lax.rem(r, PPC)
        for grp in range(D // 16):
            sl = pl.ds(grp * 16, 16)
            rows[r, sl] = rows[r, sl] * SCALE + pe[pr, sl]
        return carry

    lax.fori_loop(0, CH, row_body, 0)


def _sc_body(table_hbm, idx_hbm, pe_hbm, out_hbm, idx_v,
             r0, r1, r2, pe0, pe1, pe2,
             g0, g1, g2, q0, q1, q2, w0, w1, w2):
    c = lax.axis_index("c")
    s = lax.axis_index("s")
    wid = s * NC + c
    pos_base = wid * PPW

    pltpu.sync_copy(idx_hbm.at[wid], idx_v)

    rows = [r0, r1, r2]
    pes = [pe0, pe1, pe2]
    gsem = [g0, g1, g2]
    psem = [q0, q1, q2]
    wsem = [w0, w1, w2]

    def gather_desc(k, m):
        return pltpu.make_async_copy(
            table_hbm.at[idx_v.at[k]], rows[m], gsem[m])

    def pe_desc(k, m):
        return pltpu.make_async_copy(
            pe_hbm.at[pl.ds(pos_base + k * PPC, PPC)], pes[m], psem[m])

    def wb_descs(k, m):
        return [
            pltpu.make_async_copy(
                rows[m].at[pl.ds(b * PPC, PPC)],
                out_hbm.at[pl.ds(b * SEQ + pos_base + k * PPC, PPC)],
                wsem[m])
            for b in range(BATCH)
        ]

    def chunk_step(k, m):
        gather_desc(k, m).wait()
        pe_desc(k, m).wait()
        _fused_scale_add(rows[m], pes[m])
        for d in wb_descs(k, m):
            d.start()
        pm = (m + 2) % 3
        for d in wb_descs(k - 1, pm):
            d.wait()

        @pl.when(k + 2 < NCH)
        def _():
            gather_desc(k + 2, pm).start()
            pe_desc(k + 2, pm).start()

    gather_desc(0, 0).start()
    pe_desc(0, 0).start()
    gather_desc(1, 1).start()
    pe_desc(1, 1).start()

    gather_desc(0, 0).wait()
    pe_desc(0, 0).wait()
    _fused_scale_add(rows[0], pes[0])
    for d in wb_descs(0, 0):
        d.start()
    gather_desc(2, 2).start()
    pe_desc(2, 2).start()

    def loop_body(i, carry):
        k = 3 * i + 1
        chunk_step(k, 1)
        chunk_step(k + 1, 2)
        chunk_step(k + 2, 0)
        return carry

    lax.fori_loop(0, (NCH - 1) // 3, loop_body, 0)

    for d in wb_descs(NCH - 1, (NCH - 1) % 3):
        d.wait()


@jax.jit
def _embed(x, table):
    xp = x.astype(jnp.int32).reshape(BATCH, NW, NCH, PPC)
    idx = xp.transpose(1, 2, 0, 3).reshape(NW, NCH, CH)
    pe = jnp.asarray(_PE)
    mesh = plsc.VectorSubcoreMesh(core_axis_name="c", subcore_axis_name="s")
    out = pl.kernel(
        _sc_body,
        out_type=jax.ShapeDtypeStruct((N, D), jnp.float32),
        mesh=mesh,
        scratch_types=[
            pltpu.VMEM((NCH, CH), jnp.int32),
            pltpu.VMEM((CH, D), jnp.float32),
            pltpu.VMEM((CH, D), jnp.float32),
            pltpu.VMEM((CH, D), jnp.float32),
            pltpu.VMEM((PPC, D), jnp.float32),
            pltpu.VMEM((PPC, D), jnp.float32),
            pltpu.VMEM((PPC, D), jnp.float32),
            pltpu.SemaphoreType.DMA,
            pltpu.SemaphoreType.DMA,
            pltpu.SemaphoreType.DMA,
            pltpu.SemaphoreType.DMA,
            pltpu.SemaphoreType.DMA,
            pltpu.SemaphoreType.DMA,
            pltpu.SemaphoreType.DMA,
            pltpu.SemaphoreType.DMA,
            pltpu.SemaphoreType.DMA,
        ],
    )(table, idx, pe)
    return out.reshape(BATCH, SEQ, D)


def kernel(x, table):
    return _embed(x, table)


# speedup vs baseline: 1.1514x; 1.1514x over previous
import math

import numpy as np
import jax
import jax.numpy as jnp
from jax import lax
from jax.experimental import pallas as pl
from jax.experimental.pallas import tpu as pltpu
from jax.experimental.pallas import tpu_sc as plsc

VOCAB = 32000
D = 2048
BATCH = 4
SEQ = 2048
N = BATCH * SEQ
SCALE = math.sqrt(float(D))

NC = 2
NS = 16
NW = NC * NS
BPW = N // NW
CH = 16
NCH = BPW // CH
GRP = D // 16


def _sinusoidal_pe_np(seq_len, d_model):
    pos = np.arange(seq_len, dtype=np.float64)[:, None]
    i = np.arange(0, d_model, 2, dtype=np.float64)[None, :]
    angle = pos / np.power(10000.0, i / d_model)
    pe = np.zeros((seq_len, d_model), dtype=np.float32)
    pe[:, 0::2] = np.sin(angle)
    pe[:, 1::2] = np.cos(angle)
    return pe


_PE = _sinusoidal_pe_np(SEQ, D)


def _sc_body(table_hbm, idx_hbm, pe_hbm, out_hbm, idx_v, rows_v, out_v, pe_v,
             gsem, psem):
    c = lax.axis_index("c")
    s = lax.axis_index("s")
    wid = s * NC + c
    base = wid * BPW
    pos0 = (wid % (SEQ // BPW)) * BPW

    pltpu.sync_copy(idx_hbm.at[wid], idx_v)

    def chunk(j, carry):
        g = pltpu.async_copy(table_hbm.at[idx_v.at[j]], rows_v, gsem)
        p = pltpu.async_copy(pe_hbm.at[pl.ds(pos0 + j * CH, CH)], pe_v, psem)
        g.wait()
        p.wait()

        def row(r, carry2):
            for grp in range(GRP):
                sl = pl.ds(grp * 16, 16)
                out_v[r, sl] = rows_v[r, sl] * SCALE + pe_v[r, sl]
            return carry2

        lax.fori_loop(0, CH, row, 0)
        pltpu.sync_copy(out_v, out_hbm.at[pl.ds(base + j * CH, CH)])
        return carry

    lax.fori_loop(0, NCH, chunk, 0)


@jax.jit
def _embed(x, table):
    idx = x.reshape(N).astype(jnp.int32).reshape(NW, NCH, CH)
    pe = jnp.asarray(_PE)
    mesh = plsc.VectorSubcoreMesh(core_axis_name="c", subcore_axis_name="s")
    out = pl.kernel(
        _sc_body,
        out_type=jax.ShapeDtypeStruct((N, D), jnp.float32),
        mesh=mesh,
        scratch_types=[
            pltpu.VMEM((NCH, CH), jnp.int32),
            pltpu.VMEM((CH, D), jnp.float32),
            pltpu.VMEM((CH, D), jnp.float32),
            pltpu.VMEM((CH, D), jnp.float32),
            pltpu.SemaphoreType.DMA,
            pltpu.SemaphoreType.DMA,
        ],
    )(table, idx, pe)
    return out.reshape(BATCH, SEQ, D)


def kernel(x, table):
    return _embed(x, table)


# speedup vs baseline: 1.3083x; 1.1362x over previous
import math

import numpy as np
import jax
import jax.numpy as jnp
from jax import lax
from jax.experimental import pallas as pl
from jax.experimental.pallas import tpu as pltpu
from jax.experimental.pallas import tpu_sc as plsc

VOCAB = 32000
D = 2048
BATCH = 4
SEQ = 2048
N = BATCH * SEQ
SCALE = math.sqrt(float(D))

NC = 2
NS = 16
NW = NC * NS
BPW = N // NW
CH = 8
NCH = BPW // CH
GRP = D // 16


def _sinusoidal_pe_np(seq_len, d_model):
    pos = np.arange(seq_len, dtype=np.float64)[:, None]
    i = np.arange(0, d_model, 2, dtype=np.float64)[None, :]
    angle = pos / np.power(10000.0, i / d_model)
    pe = np.zeros((seq_len, d_model), dtype=np.float32)
    pe[:, 0::2] = np.sin(angle)
    pe[:, 1::2] = np.cos(angle)
    return pe


_PE = _sinusoidal_pe_np(SEQ, D)


def _sc_body(table_hbm, idx_hbm, pe_hbm, out_hbm, idx_v,
             rA, rB, pA, pB, gsA, gsB, psA, psB):
    c = lax.axis_index("c")
    s = lax.axis_index("s")
    wid = s * NC + c
    base = wid * BPW
    pos0 = (wid % (SEQ // BPW)) * BPW

    pltpu.sync_copy(idx_hbm.at[wid], idx_v)

    rows = [rA, rB]
    pes = [pA, pB]
    gsem = [gsA, gsB]
    psem = [psA, psB]

    def g_desc(k, m):
        return pltpu.make_async_copy(
            table_hbm.at[idx_v.at[k]], rows[m], gsem[m])

    def p_desc(k, m):
        return pltpu.make_async_copy(
            pe_hbm.at[pl.ds(pos0 + k * CH, CH)], pes[m], psem[m])

    def compute(rv, pv):
        def row(r, carry):
            for grp in range(GRP):
                sl = pl.ds(grp * 16, 16)
                rv[r, sl] = rv[r, sl] * SCALE + pv[r, sl]
            return carry

        lax.fori_loop(0, CH, row, 0)

    def consume(k, m):
        g_desc(k, m).wait()
        p_desc(k, m).wait()
        compute(rows[m], pes[m])
        pltpu.sync_copy(rows[m], out_hbm.at[pl.ds(base + k * CH, CH)])

    g_desc(0, 0).start()
    p_desc(0, 0).start()

    def pair(i, carry):
        a = 2 * i
        g_desc(a + 1, 1).start()
        p_desc(a + 1, 1).start()
        consume(a, 0)
        g_desc(a + 2, 0).start()
        p_desc(a + 2, 0).start()
        consume(a + 1, 1)
        return carry

    lax.fori_loop(0, NCH // 2 - 1, pair, 0)

    g_desc(NCH - 1, 1).start()
    p_desc(NCH - 1, 1).start()
    consume(NCH - 2, 0)
    consume(NCH - 1, 1)


@jax.jit
def _embed(x, table):
    idx = x.reshape(N).astype(jnp.int32).reshape(NW, NCH, CH)
    pe = jnp.asarray(_PE)
    mesh = plsc.VectorSubcoreMesh(core_axis_name="c", subcore_axis_name="s")
    out = pl.kernel(
        _sc_body,
        out_type=jax.ShapeDtypeStruct((N, D), jnp.float32),
        mesh=mesh,
        scratch_types=[
            pltpu.VMEM((NCH, CH), jnp.int32),
            pltpu.VMEM((CH, D), jnp.float32),
            pltpu.VMEM((CH, D), jnp.float32),
            pltpu.VMEM((CH, D), jnp.float32),
            pltpu.VMEM((CH, D), jnp.float32),
            pltpu.SemaphoreType.DMA,
            pltpu.SemaphoreType.DMA,
            pltpu.SemaphoreType.DMA,
            pltpu.SemaphoreType.DMA,
        ],
    )(table, idx, pe)
    return out.reshape(BATCH, SEQ, D)


def kernel(x, table):
    return _embed(x, table)


# speedup vs baseline: 1.4396x; 1.1004x over previous
import math

import numpy as np
import jax
import jax.numpy as jnp
from jax import lax
from jax.experimental import pallas as pl
from jax.experimental.pallas import tpu as pltpu
from jax.experimental.pallas import tpu_sc as plsc

VOCAB = 32000
D = 2048
BATCH = 4
SEQ = 2048
N = BATCH * SEQ
SCALE = math.sqrt(float(D))

NC = 2
NS = 16
NW = NC * NS
BPW = N // NW
CH = 8
NCH = BPW // CH
GRP = D // 16


def _sinusoidal_pe_np(seq_len, d_model):
    pos = np.arange(seq_len, dtype=np.float64)[:, None]
    i = np.arange(0, d_model, 2, dtype=np.float64)[None, :]
    angle = pos / np.power(10000.0, i / d_model)
    pe = np.zeros((seq_len, d_model), dtype=np.float32)
    pe[:, 0::2] = np.sin(angle)
    pe[:, 1::2] = np.cos(angle)
    return pe


_PE = _sinusoidal_pe_np(SEQ, D)


def _sc_body(table_hbm, idx_hbm, pe_hbm, out_hbm, idx_v,
             r0, r1, r2, p0, p1, p2,
             g0, g1, g2, q0, q1, q2, w0, w1, w2):
    c = lax.axis_index("c")
    s = lax.axis_index("s")
    wid = s * NC + c
    base = wid * BPW
    pos0 = (wid % (SEQ // BPW)) * BPW

    pltpu.sync_copy(idx_hbm.at[wid], idx_v)

    rows = [r0, r1, r2]
    pes = [p0, p1, p2]
    gsem = [g0, g1, g2]
    psem = [q0, q1, q2]
    wsem = [w0, w1, w2]

    def g_desc(k, m):
        return pltpu.make_async_copy(
            table_hbm.at[idx_v.at[k]], rows[m], gsem[m])

    def p_desc(k, m):
        return pltpu.make_async_copy(
            pe_hbm.at[pl.ds(pos0 + k * CH, CH)], pes[m], psem[m])

    def w_desc(k, m):
        return pltpu.make_async_copy(
            rows[m], out_hbm.at[pl.ds(base + k * CH, CH)], wsem[m])

    def compute(rv, pv):
        def row(r, carry):
            for grp in range(GRP):
                sl = pl.ds(grp * 16, 16)
                rv[r, sl] = rv[r, sl] * SCALE + pv[r, sl]
            return carry

        lax.fori_loop(0, CH, row, 0)

    def consume(k, m, drain_prev=True, prefetch=True):
        g_desc(k, m).wait()
        p_desc(k, m).wait()
        compute(rows[m], pes[m])
        w_desc(k, m).start()
        nm = (m + 2) % 3
        if drain_prev:
            w_desc(k - 1, nm).wait()
        if prefetch:
            g_desc(k + 2, nm).start()
            p_desc(k + 2, nm).start()

    g_desc(0, 0).start()
    p_desc(0, 0).start()
    g_desc(1, 1).start()
    p_desc(1, 1).start()

    consume(0, 0, drain_prev=False)

    def loop_body(i, carry):
        k = 3 * i + 1
        consume(k, 1)
        consume(k + 1, 2)
        consume(k + 2, 0)
        return carry

    lax.fori_loop(0, 9, loop_body, 0)

    consume(28, 1)
    consume(29, 2)
    consume(30, 0, prefetch=False)
    consume(31, 1, prefetch=False)

    w_desc(31, 1).wait()


@jax.jit
def _embed(x, table):
    idx = x.reshape(N).astype(jnp.int32).reshape(NW, NCH, CH)
    pe = jnp.asarray(_PE)
    mesh = plsc.VectorSubcoreMesh(core_axis_name="c", subcore_axis_name="s")
    out = pl.kernel(
        _sc_body,
        out_type=jax.ShapeDtypeStruct((N, D), jnp.float32),
        mesh=mesh,
        scratch_types=[
            pltpu.VMEM((NCH, CH), jnp.int32),
            pltpu.VMEM((CH, D), jnp.float32),
            pltpu.VMEM((CH, D), jnp.float32),
            pltpu.VMEM((CH, D), jnp.float32),
            pltpu.VMEM((CH, D), jnp.float32),
            pltpu.VMEM((CH, D), jnp.float32),
            pltpu.VMEM((CH, D), jnp.float32),
            pltpu.SemaphoreType.DMA,
            pltpu.SemaphoreType.DMA,
            pltpu.SemaphoreType.DMA,
            pltpu.SemaphoreType.DMA,
            pltpu.SemaphoreType.DMA,
            pltpu.SemaphoreType.DMA,
            pltpu.SemaphoreType.DMA,
            pltpu.SemaphoreType.DMA,
            pltpu.SemaphoreType.DMA,
        ],
    )(table, idx, pe)
    return out.reshape(BATCH, SEQ, D)


def kernel(x, table):
    return _embed(x, table)
